# baseline (speedup 1.0000x reference)
import functools

import jax
import jax.numpy as jnp
from jax.experimental import pallas as pl
from jax.experimental.pallas import tpu as pltpu

B, D, S = 4, 1024, 2048
K = 256
_ROWS_PER_STEP = 256


def _mean_kernel(w_ref, out_ref, acc_ref):
    j = pl.program_id(1)
    nj = pl.num_programs(1)

    @pl.when(j == 0)
    def _init():
        acc_ref[...] = jnp.zeros_like(acc_ref)

    acc = acc_ref[...]
    for g in range(_ROWS_PER_STEP // 8):
        acc = acc + w_ref[0, 8 * g:8 * g + 8, :]
    acc_ref[...] = acc

    @pl.when(j == nj - 1)
    def _finish():
        a = acc_ref[...]
        t = a[0:4] + a[4:8]
        u = t[0:2] + t[2:4]
        s = u[0:1] + u[1:2]
        out_ref[0] = s * (1.0 / S)


def _topk_kernel(m_ref, idx_ref):
    a0 = m_ref[...]
    iota = jax.lax.broadcasted_iota(jnp.int32, (B, S), 1)
    kio = jax.lax.broadcasted_iota(jnp.int32, (B, K), 1)

    def body(k, carry):
        a, out = carry
        mx = jnp.max(a, axis=1, keepdims=True)
        sel = jnp.where(a == mx, iota, S)
        j = jnp.min(sel, axis=1, keepdims=True)
        out = jnp.where(kio == k, j, out)
        a = jnp.where(iota == j, -jnp.inf, a)
        return a, out

    _, out = jax.lax.fori_loop(
        0, K, body, (a0, jnp.zeros((B, K), jnp.int32)))
    idx_ref[...] = out


def _gather_kernel(x_ref, idx_ref, out_ref):
    idx_row = idx_ref[0]
    onehot = (jax.lax.broadcasted_iota(jnp.int32, (S, K), 0)
              == idx_row).astype(jnp.float32)
    out_ref[0] = jnp.dot(x_ref[0], onehot,
                         preferred_element_type=jnp.float32,
                         precision=jax.lax.Precision.HIGHEST)


@functools.partial(jax.jit)
def kernel(x, w):
    nsteps = S // _ROWS_PER_STEP
    w_mean = pl.pallas_call(
        _mean_kernel,
        grid=(B, nsteps),
        in_specs=[pl.BlockSpec((1, _ROWS_PER_STEP, S),
                               lambda b, j: (b, j, 0))],
        out_specs=pl.BlockSpec((1, 1, S), lambda b, j: (b, 0, 0)),
        out_shape=jax.ShapeDtypeStruct((B, 1, S), jnp.float32),
        scratch_shapes=[pltpu.VMEM((8, S), jnp.float32)],
        compiler_params=pltpu.CompilerParams(
            dimension_semantics=("arbitrary", "arbitrary")),
    )(w)

    idx = pl.pallas_call(
        _topk_kernel,
        out_shape=jax.ShapeDtypeStruct((B, K), jnp.int32),
    )(w_mean.reshape(B, S))

    idx3 = idx.reshape(B, 1, K)
    out = pl.pallas_call(
        _gather_kernel,
        grid=(B,),
        in_specs=[
            pl.BlockSpec((1, D, S), lambda b: (b, 0, 0)),
            pl.BlockSpec((1, 1, K), lambda b: (b, 0, 0)),
        ],
        out_specs=pl.BlockSpec((1, D, K), lambda b: (b, 0, 0)),
        out_shape=jax.ShapeDtypeStruct((B, D, K), jnp.float32),
        compiler_params=pltpu.CompilerParams(
            dimension_semantics=("arbitrary",)),
    )(x, idx3)
    return out
